# Initial kernel scaffold; baseline (speedup 1.0000x reference)
import jax
import jax.numpy as jnp
from jax.experimental import pallas as pl


def kernel(states, edge_index, x_init, phi_W1, phi_b1, phi_W2, phi_b2, phi_W3, phi_b3, lstm_Wih0, lstm_Whh0, lstm_bih0, lstm_bhh0, lstm_Wih1, lstm_Whh1, lstm_bih1, lstm_bhh1, out_W, out_b):
    raise NotImplementedError("write your pallas kernel here")


# speedup vs baseline: 6.7951x; 6.7951x over previous
import functools

import jax
import jax.numpy as jnp
from jax import lax
from jax.experimental import pallas as pl
from jax.experimental.pallas import tpu as pltpu
from jax.experimental.pallas import tpu_sc as plsc

N = 10000
FEAT = 128
STATE = 16
H = 128
HID = 32
CNT_LANE = 20
NW = 32
CHUNK = 128
N_PAD = 10240

f32 = jnp.float32


def _tables_body(x_ref, s_ref, w1x_ref, w1s_ref, b1_ref, a_ref, b_ref, *, relu_x):
    x = x_ref[...]
    if relu_x:
        x = jnp.maximum(x, 0.0)
    sb = jnp.dot(s_ref[...], w1s_ref[...], preferred_element_type=f32)
    a_ref[...] = jnp.dot(x, w1x_ref[...], preferred_element_type=f32) + sb + b1_ref[...]
    b_ref[...] = sb


def _tables_call(x_p, states_p, w1x, w1s, b1p, relu_x):
    blk = 2560
    grid = (N_PAD // blk,)
    return pl.pallas_call(
        functools.partial(_tables_body, relu_x=relu_x),
        grid=grid,
        in_specs=[
            pl.BlockSpec((blk, FEAT), lambda i: (i, 0)),
            pl.BlockSpec((blk, STATE), lambda i: (i, 0)),
            pl.BlockSpec((FEAT, HID), lambda i: (0, 0)),
            pl.BlockSpec((STATE, HID), lambda i: (0, 0)),
            pl.BlockSpec((1, HID), lambda i: (0, 0)),
        ],
        out_specs=[
            pl.BlockSpec((blk, HID), lambda i: (i, 0)),
            pl.BlockSpec((blk, HID), lambda i: (i, 0)),
        ],
        out_shape=[
            jax.ShapeDtypeStruct((N_PAD, HID), f32),
            jax.ShapeDtypeStruct((N_PAD, HID), f32),
        ],
    )(x_p, states_p, w1x, w1s, b1p)


def _edge_mlp_body(g1_ref, g2_ref, w2_ref, b2_ref, out_ref):
    h1 = jnp.maximum(g1_ref[...] - g2_ref[...], 0.0)
    h2 = jnp.maximum(
        jnp.dot(h1, w2_ref[...], preferred_element_type=f32) + b2_ref[...], 0.0)
    lane = lax.broadcasted_iota(jnp.int32, h2.shape, 1)
    out_ref[...] = h2 + jnp.where(lane == CNT_LANE, 1.0, 0.0)


def _edge_mlp_call(g1, g2, w2p, b2p):
    e_pad = g1.shape[0]
    blk = 4096
    grid = (e_pad // blk,)
    return pl.pallas_call(
        _edge_mlp_body,
        grid=grid,
        in_specs=[
            pl.BlockSpec((blk, HID), lambda i: (i, 0)),
            pl.BlockSpec((blk, HID), lambda i: (i, 0)),
            pl.BlockSpec((HID, HID), lambda i: (0, 0)),
            pl.BlockSpec((1, HID), lambda i: (0, 0)),
        ],
        out_specs=pl.BlockSpec((blk, HID), lambda i: (i, 0)),
        out_shape=jax.ShapeDtypeStruct((e_pad, HID), f32),
    )(g1, g2, w2p, b2p)


def _pre_body(s0_ref, s1_ref, x_ref, w3_ref, b3_ref, wta_ref, wtx_ref, b0_ref,
              pre_ref, *, relu_x):
    s = s0_ref[...] + s1_ref[...]
    lane = lax.broadcasted_iota(jnp.int32, s.shape, 1)
    cnt = jnp.sum(jnp.where(lane == CNT_LANE, s, 0.0), axis=1, keepdims=True)
    inv = 1.0 / jnp.maximum(cnt, 1.0)
    raw = jnp.dot(s, w3_ref[...], preferred_element_type=f32)
    aggr = raw * inv + b3_ref[...] * (cnt * inv)
    x = x_ref[...]
    if relu_x:
        x = jnp.maximum(x, 0.0)
    pre_ref[...] = (jnp.dot(aggr, wta_ref[...], preferred_element_type=f32)
                    + jnp.dot(x, wtx_ref[...], preferred_element_type=f32)
                    + b0_ref[...])


def _pre_call(s0, s1, x, w3p, b3p, wta, wtx, b0s, relu_x):
    blk = 2000
    grid = (N // blk,)
    return pl.pallas_call(
        functools.partial(_pre_body, relu_x=relu_x),
        grid=grid,
        in_specs=[
            pl.BlockSpec((blk, HID), lambda i: (i, 0)),
            pl.BlockSpec((blk, HID), lambda i: (i, 0)),
            pl.BlockSpec((blk, FEAT), lambda i: (i, 0)),
            pl.BlockSpec((HID, FEAT), lambda i: (0, 0)),
            pl.BlockSpec((1, FEAT), lambda i: (0, 0)),
            pl.BlockSpec((FEAT, 4 * H), lambda i: (0, 0)),
            pl.BlockSpec((FEAT, 4 * H), lambda i: (0, 0)),
            pl.BlockSpec((1, 4 * H), lambda i: (0, 0)),
        ],
        out_specs=pl.BlockSpec((blk, 4 * H), lambda i: (i, 0)),
        out_shape=jax.ShapeDtypeStruct((N, 4 * H), f32),
    )(s0, s1, x, w3p, b3p, wta, wtx, b0s)


def _scan_body(pre_ref, wh0_ref, wi1_ref, wh1_ref, b1_ref, out_ref,
               h0s, c0s, h1s, c1s, *, rows):
    @pl.when(pl.program_id(0) == 0)
    def _init():
        z = jnp.zeros((8, H), f32)
        h0s[...] = z
        c0s[...] = z
        h1s[...] = z
        c1s[...] = z

    wh0 = wh0_ref[...]
    wi1 = wi1_ref[...]
    wh1 = wh1_ref[...]
    b1 = b1_ref[...]

    def step(t, carry):
        h0, c0, h1, c1 = carry
        g0 = pre_ref[pl.ds(t, 1), :] + jnp.dot(h0, wh0, preferred_element_type=f32)
        c0 = (jax.nn.sigmoid(g0[:, H:2 * H]) * c0
              + jax.nn.sigmoid(g0[:, :H]) * jnp.tanh(g0[:, 2 * H:3 * H]))
        h0 = jax.nn.sigmoid(g0[:, 3 * H:]) * jnp.tanh(c0)
        g1 = (jnp.dot(h0, wi1, preferred_element_type=f32) + b1
              + jnp.dot(h1, wh1, preferred_element_type=f32))
        c1 = (jax.nn.sigmoid(g1[:, H:2 * H]) * c1
              + jax.nn.sigmoid(g1[:, :H]) * jnp.tanh(g1[:, 2 * H:3 * H]))
        h1 = jax.nn.sigmoid(g1[:, 3 * H:]) * jnp.tanh(c1)
        out_ref[pl.ds(t, 1), :] = h1
        return (h0, c0, h1, c1)

    carry = (h0s[0:1, :], c0s[0:1, :], h1s[0:1, :], c1s[0:1, :])
    h0, c0, h1, c1 = lax.fori_loop(0, rows, step, carry)
    h0s[0:1, :] = h0
    c0s[0:1, :] = c0
    h1s[0:1, :] = h1
    c1s[0:1, :] = c1


def _scan_call(pre0, wh0, wi1, wh1, b1s):
    blk = 1000
    grid = (N // blk,)
    return pl.pallas_call(
        functools.partial(_scan_body, rows=blk),
        grid=grid,
        in_specs=[
            pl.BlockSpec((blk, 4 * H), lambda i: (i, 0)),
            pl.BlockSpec((H, 4 * H), lambda i: (0, 0)),
            pl.BlockSpec((H, 4 * H), lambda i: (0, 0)),
            pl.BlockSpec((H, 4 * H), lambda i: (0, 0)),
            pl.BlockSpec((1, 4 * H), lambda i: (0, 0)),
        ],
        out_specs=pl.BlockSpec((blk, H), lambda i: (i, 0)),
        out_shape=jax.ShapeDtypeStruct((N, H), f32),
        scratch_shapes=[pltpu.VMEM((8, H), f32) for _ in range(4)],
    )(pre0, wh0, wi1, wh1, b1s)


def _head_body(x_ref, w_ref, b_ref, out_ref):
    x = jnp.maximum(x_ref[...], 0.0)
    out_ref[...] = jnp.dot(x, w_ref[...], preferred_element_type=f32) + b_ref[...]


def _head_call(x, wp, bp):
    blk = 2000
    grid = (N // blk,)
    return pl.pallas_call(
        _head_body,
        grid=grid,
        in_specs=[
            pl.BlockSpec((blk, FEAT), lambda i: (i, 0)),
            pl.BlockSpec((FEAT, FEAT), lambda i: (0, 0)),
            pl.BlockSpec((1, FEAT), lambda i: (0, 0)),
        ],
        out_specs=pl.BlockSpec((blk, FEAT), lambda i: (i, 0)),
        out_shape=jax.ShapeDtypeStruct((N, FEAT), f32),
    )(x, wp, bp)


def _make_gather(e_pad, nch):
    mesh = plsc.VectorSubcoreMesh(core_axis_name="c", subcore_axis_name="s")

    @functools.partial(
        pl.kernel,
        mesh=mesh,
        out_type=[
            jax.ShapeDtypeStruct((e_pad, HID), f32),
            jax.ShapeDtypeStruct((e_pad, HID), f32),
        ],
        scratch_types=[
            pltpu.VMEM((nch, CHUNK), jnp.int32),
            pltpu.VMEM((nch, CHUNK), jnp.int32),
            pltpu.VMEM((CHUNK, HID), f32),
            pltpu.VMEM((CHUNK, HID), f32),
            pltpu.SemaphoreType.DMA,
            pltpu.SemaphoreType.DMA,
        ],
        compiler_params=pltpu.CompilerParams(use_tc_tiling_on_sc=False),
    )
    def gather_k(a_hbm, b_hbm, src_hbm, dst_hbm, g1_hbm, g2_hbm,
                 sidx, didx, r1, r2, sem1, sem2):
        wid = lax.axis_index("s") * 2 + lax.axis_index("c")
        pltpu.sync_copy(src_hbm.at[wid], sidx)
        pltpu.sync_copy(dst_hbm.at[wid], didx)
        base = wid * (nch * CHUNK)

        def body(j, carry):
            cp1 = pltpu.async_copy(a_hbm.at[sidx.at[j]], r1, sem1)
            cp2 = pltpu.async_copy(b_hbm.at[didx.at[j]], r2, sem2)
            cp1.wait()
            cp2.wait()
            pltpu.sync_copy(r1, g1_hbm.at[pl.ds(base + j * CHUNK, CHUNK)])
            pltpu.sync_copy(r2, g2_hbm.at[pl.ds(base + j * CHUNK, CHUNK)])
            return carry

        lax.fori_loop(0, nch, body, 0)

    return gather_k


def _make_scatter(nch):
    mesh = plsc.VectorSubcoreMesh(core_axis_name="c", subcore_axis_name="s")
    rows_t = N_PAD // 16

    @functools.partial(
        pl.kernel,
        mesh=mesh,
        out_type=jax.ShapeDtypeStruct((2, N_PAD, HID), f32),
        scratch_types=[
            pltpu.VMEM((nch, CHUNK), jnp.int32),
            pltpu.VMEM((CHUNK, HID), f32),
            pltpu.VMEM_SHARED((N_PAD, HID), f32),
        ],
        compiler_params=pltpu.CompilerParams(use_tc_tiling_on_sc=False),
    )
    def scatter_k(h2_hbm, dst_hbm, zero_hbm, out_hbm, didx, dat, acc):
        cid = lax.axis_index("c")
        sid = lax.axis_index("s")
        wid = sid * 2 + cid
        pltpu.sync_copy(zero_hbm.at[pl.ds(sid * rows_t, rows_t)],
                        acc.at[pl.ds(sid * rows_t, rows_t)])
        pltpu.sync_copy(dst_hbm.at[wid], didx)
        plsc.subcore_barrier()
        base = wid * (nch * CHUNK)

        def body(j, carry):
            pltpu.sync_copy(h2_hbm.at[pl.ds(base + j * CHUNK, CHUNK)], dat)
            pltpu.sync_copy(dat, acc.at[didx.at[j]], add=True)
            return carry

        lax.fori_loop(0, nch, body, 0)
        plsc.subcore_barrier()
        pltpu.sync_copy(acc.at[pl.ds(sid * rows_t, rows_t)],
                        out_hbm.at[cid, pl.ds(sid * rows_t, rows_t)])

    return scatter_k


def kernel(states, edge_index, x_init, phi_W1, phi_b1, phi_W2, phi_b2, phi_W3,
           phi_b3, lstm_Wih0, lstm_Whh0, lstm_bih0, lstm_bhh0,
           lstm_Wih1, lstm_Whh1, lstm_bih1, lstm_bhh1, out_W, out_b):
    e = edge_index.shape[1]
    nch = -(-e // (NW * CHUNK))
    e_pad = NW * nch * CHUNK

    pad_idx = jnp.full((e_pad - e,), N, jnp.int32)
    srcw = jnp.concatenate([edge_index[0], pad_idx]).reshape(NW, nch, CHUNK)
    dstw = jnp.concatenate([edge_index[1], pad_idx]).reshape(NW, nch, CHUNK)
    zeros_tab = jnp.zeros((N_PAD, HID), f32)
    states_p = jnp.pad(states, ((0, N_PAD - N), (0, 0)))

    gather_k = _make_gather(e_pad, nch)
    scatter_k = _make_scatter(nch)

    x = x_init
    for l in range(2):
        relu_x = (l == 1)
        w1 = phi_W1[l]
        w1x = jnp.pad(w1[:FEAT], ((0, 0), (0, HID - 20)))
        w1s = jnp.pad(w1[FEAT:], ((0, 0), (0, HID - 20)))
        b1p = jnp.pad(phi_b1[l], (0, HID - 20)).reshape(1, HID)
        w2p = jnp.pad(phi_W2[l], ((0, HID - 20), (0, HID - 20)))
        b2p = jnp.pad(phi_b2[l], (0, HID - 20)).reshape(1, HID)
        w3p = jnp.pad(phi_W3[l], ((0, HID - 20), (0, 0)))
        b3p = phi_b3[l].reshape(1, FEAT)
        wih0t = lstm_Wih0[l].T
        wta = wih0t[:FEAT]
        wtx = wih0t[FEAT:]
        b0s = (lstm_bih0[l] + lstm_bhh0[l]).reshape(1, 4 * H)
        wh0 = lstm_Whh0[l].T
        wi1 = lstm_Wih1[l].T
        wh1 = lstm_Whh1[l].T
        b1s = (lstm_bih1[l] + lstm_bhh1[l]).reshape(1, 4 * H)

        x_p = jnp.pad(x, ((0, N_PAD - N), (0, 0)))
        a_t, b_t = _tables_call(x_p, states_p, w1x, w1s, b1p, relu_x)
        g1, g2 = gather_k(a_t, b_t, srcw, dstw)
        h2 = _edge_mlp_call(g1, g2, w2p, b2p)
        s2 = scatter_k(h2, dstw, zeros_tab)
        pre0 = _pre_call(s2[0, :N], s2[1, :N], x, w3p, b3p, wta, wtx, b0s,
                         relu_x)
        x = _scan_call(pre0, wh0, wi1, wh1, b1s)

    wp = jnp.pad(out_W, ((0, 0), (0, FEAT - 1)))
    bp = jnp.broadcast_to(out_b.reshape(1, 1), (1, FEAT))
    y = _head_call(x, wp, bp)
    return y[:, :1]


# speedup vs baseline: 8.0209x; 1.1804x over previous
import functools

import jax
import jax.numpy as jnp
from jax import lax
from jax.experimental import pallas as pl
from jax.experimental.pallas import tpu as pltpu
from jax.experimental.pallas import tpu_sc as plsc

N = 10000
FEAT = 128
STATE = 16
H = 128
HID = 32
MSG = 144
NW = 32
CHUNK = 128
N_PAD = 10240

f32 = jnp.float32
bf16 = jnp.bfloat16


def _b(x):
    return x.astype(bf16)


def _tables_body(x_ref, w1x_ref, a_ref, *, relu_x):
    x = x_ref[...]
    if relu_x:
        x = jnp.maximum(x, 0.0)
    a_ref[...] = jnp.dot(_b(x), w1x_ref[...], preferred_element_type=f32)


def _tables_call(x_p, w1x, relu_x):
    blk = 2560
    grid = (N_PAD // blk,)
    return pl.pallas_call(
        functools.partial(_tables_body, relu_x=relu_x),
        grid=grid,
        in_specs=[
            pl.BlockSpec((blk, FEAT), lambda i: (i, 0)),
            pl.BlockSpec((FEAT, HID), lambda i: (0, 0)),
        ],
        out_specs=pl.BlockSpec((blk, HID), lambda i: (i, 0)),
        out_shape=jax.ShapeDtypeStruct((N_PAD, HID), f32),
    )(x_p, _b(w1x))


def _edge_mlp_body(ax_ref, ss_ref, sd_ref, w1s_ref, b1_ref, w2_ref, b2_ref,
                   w3_ref, out_ref):
    sdiff = ss_ref[...] - sd_ref[...]
    h1 = jnp.maximum(
        ax_ref[...]
        + jnp.dot(_b(sdiff), w1s_ref[...], preferred_element_type=f32)
        + b1_ref[...], 0.0)
    h2 = jnp.maximum(
        jnp.dot(_b(h1), w2_ref[...], preferred_element_type=f32)
        + b2_ref[...], 0.0)
    msg = jnp.dot(_b(h2), w3_ref[...], preferred_element_type=f32)
    lane = lax.broadcasted_iota(jnp.int32, (msg.shape[0], MSG - FEAT), 1)
    tail = jnp.where(lane == 0, 1.0, 0.0)
    out_ref[...] = jnp.concatenate([msg, tail], axis=1)


def _edge_mlp_call(g1, g2, g3, w1s, b1p, w2p, b2p, w3p):
    e_pad = g1.shape[0]
    blk = 4096
    grid = (e_pad // blk,)
    return pl.pallas_call(
        _edge_mlp_body,
        grid=grid,
        in_specs=[
            pl.BlockSpec((blk, HID), lambda i: (i, 0)),
            pl.BlockSpec((blk, STATE), lambda i: (i, 0)),
            pl.BlockSpec((blk, STATE), lambda i: (i, 0)),
            pl.BlockSpec((STATE, HID), lambda i: (0, 0)),
            pl.BlockSpec((1, HID), lambda i: (0, 0)),
            pl.BlockSpec((HID, HID), lambda i: (0, 0)),
            pl.BlockSpec((1, HID), lambda i: (0, 0)),
            pl.BlockSpec((HID, FEAT), lambda i: (0, 0)),
        ],
        out_specs=pl.BlockSpec((blk, MSG), lambda i: (i, 0)),
        out_shape=jax.ShapeDtypeStruct((e_pad, MSG), f32),
    )(g1, g2, g3, _b(w1s), b1p, _b(w2p), b2p, _b(w3p))


def _pre_body(s0_ref, s1_ref, x_ref, b3_ref, wta_ref, wtx_ref, b0_ref,
              pre_ref, *, relu_x):
    s = s0_ref[...] + s1_ref[...]
    cnt = s[:, FEAT:FEAT + 1]
    summ = s[:, :FEAT] + b3_ref[...] * cnt
    aggr = summ / jnp.maximum(cnt, 1.0)
    x = x_ref[...]
    if relu_x:
        x = jnp.maximum(x, 0.0)
    pre_ref[...] = (jnp.dot(_b(aggr), wta_ref[...], preferred_element_type=f32)
                    + jnp.dot(_b(x), wtx_ref[...], preferred_element_type=f32)
                    + b0_ref[...])


def _pre_call(s0, s1, x, b3p, wta, wtx, b0s, relu_x):
    blk = 2000
    grid = (N // blk,)
    return pl.pallas_call(
        functools.partial(_pre_body, relu_x=relu_x),
        grid=grid,
        in_specs=[
            pl.BlockSpec((blk, MSG), lambda i: (i, 0)),
            pl.BlockSpec((blk, MSG), lambda i: (i, 0)),
            pl.BlockSpec((blk, FEAT), lambda i: (i, 0)),
            pl.BlockSpec((1, FEAT), lambda i: (0, 0)),
            pl.BlockSpec((FEAT, 4 * H), lambda i: (0, 0)),
            pl.BlockSpec((FEAT, 4 * H), lambda i: (0, 0)),
            pl.BlockSpec((1, 4 * H), lambda i: (0, 0)),
        ],
        out_specs=pl.BlockSpec((blk, 4 * H), lambda i: (i, 0)),
        out_shape=jax.ShapeDtypeStruct((N, 4 * H), f32),
    )(s0, s1, x, b3p, _b(wta), _b(wtx), b0s)


def _scan_body(pre_ref, w01_ref, wh1_ref, b1_ref, out_ref,
               h0s, c0s, h1s, c1s, *, rows):
    @pl.when(pl.program_id(0) == 0)
    def _init():
        z = jnp.zeros((8, H), f32)
        h0s[...] = z
        c0s[...] = z
        h1s[...] = z
        c1s[...] = z

    b1 = b1_ref[...]

    def step(t, carry):
        h0, c0, h1, c1 = carry
        tt = jnp.minimum(t, rows - 1)
        p = pre_ref[pl.ds(tt, 1), :]
        z01 = jnp.dot(_b(h0), w01_ref[...], preferred_element_type=f32)
        zh1 = jnp.dot(_b(h1), wh1_ref[...], preferred_element_type=f32)
        g0 = p + z01[:, :4 * H]
        g1 = z01[:, 4 * H:] + zh1 + b1
        c0n = (jax.nn.sigmoid(g0[:, H:2 * H]) * c0
               + jax.nn.sigmoid(g0[:, :H]) * jnp.tanh(g0[:, 2 * H:3 * H]))
        h0n = jax.nn.sigmoid(g0[:, 3 * H:]) * jnp.tanh(c0n)
        c1n = (jax.nn.sigmoid(g1[:, H:2 * H]) * c1
               + jax.nn.sigmoid(g1[:, :H]) * jnp.tanh(g1[:, 2 * H:3 * H]))
        h1n = jax.nn.sigmoid(g1[:, 3 * H:]) * jnp.tanh(c1n)

        @pl.when(t > 0)
        def _store():
            out_ref[pl.ds(t - 1, 1), :] = h1n

        lead = t < rows
        trail = t > 0
        h0 = jnp.where(lead, h0n, h0)
        c0 = jnp.where(lead, c0n, c0)
        h1 = jnp.where(trail, h1n, h1)
        c1 = jnp.where(trail, c1n, c1)
        return (h0, c0, h1, c1)

    carry = (h0s[0:1, :], c0s[0:1, :], h1s[0:1, :], c1s[0:1, :])
    h0, c0, h1, c1 = lax.fori_loop(0, rows + 1, step, carry, unroll=2)
    h0s[0:1, :] = h0
    c0s[0:1, :] = c0
    h1s[0:1, :] = h1
    c1s[0:1, :] = c1


def _scan_call(pre0, wh0, wi1, wh1, b1s):
    blk = 1000
    grid = (N // blk,)
    w01 = _b(jnp.concatenate([wh0, wi1], axis=1))
    return pl.pallas_call(
        functools.partial(_scan_body, rows=blk),
        grid=grid,
        in_specs=[
            pl.BlockSpec((blk, 4 * H), lambda i: (i, 0)),
            pl.BlockSpec((H, 8 * H), lambda i: (0, 0)),
            pl.BlockSpec((H, 4 * H), lambda i: (0, 0)),
            pl.BlockSpec((1, 4 * H), lambda i: (0, 0)),
        ],
        out_specs=pl.BlockSpec((blk, H), lambda i: (i, 0)),
        out_shape=jax.ShapeDtypeStruct((N, H), f32),
        scratch_shapes=[pltpu.VMEM((8, H), f32) for _ in range(4)],
    )(pre0, w01, _b(wh1), b1s)


def _head_body(x_ref, w_ref, b_ref, out_ref):
    x = jnp.maximum(x_ref[...], 0.0)
    out_ref[...] = (jnp.dot(_b(x), w_ref[...], preferred_element_type=f32)
                    + b_ref[...])


def _head_call(x, wp, bp):
    blk = 2000
    grid = (N // blk,)
    return pl.pallas_call(
        _head_body,
        grid=grid,
        in_specs=[
            pl.BlockSpec((blk, FEAT), lambda i: (i, 0)),
            pl.BlockSpec((FEAT, FEAT), lambda i: (0, 0)),
            pl.BlockSpec((1, FEAT), lambda i: (0, 0)),
        ],
        out_specs=pl.BlockSpec((blk, FEAT), lambda i: (i, 0)),
        out_shape=jax.ShapeDtypeStruct((N, FEAT), f32),
    )(x, _b(wp), bp)


def _make_gather(e_pad, nch):
    mesh = plsc.VectorSubcoreMesh(core_axis_name="c", subcore_axis_name="s")

    @functools.partial(
        pl.kernel,
        mesh=mesh,
        out_type=[
            jax.ShapeDtypeStruct((e_pad, HID), f32),
            jax.ShapeDtypeStruct((e_pad, STATE), f32),
            jax.ShapeDtypeStruct((e_pad, STATE), f32),
        ],
        scratch_types=[
            pltpu.VMEM((nch, CHUNK), jnp.int32),
            pltpu.VMEM((nch, CHUNK), jnp.int32),
            pltpu.VMEM((CHUNK, HID), f32),
            pltpu.VMEM((CHUNK, STATE), f32),
            pltpu.VMEM((CHUNK, STATE), f32),
            pltpu.SemaphoreType.DMA,
            pltpu.SemaphoreType.DMA,
            pltpu.SemaphoreType.DMA,
        ],
        compiler_params=pltpu.CompilerParams(use_tc_tiling_on_sc=False),
    )
    def gather_k(a_hbm, st_hbm, src_hbm, dst_hbm, g1_hbm, g2_hbm, g3_hbm,
                 sidx, didx, r1, r2, r3, sem1, sem2, sem3):
        wid = lax.axis_index("s") * 2 + lax.axis_index("c")
        pltpu.sync_copy(src_hbm.at[wid], sidx)
        pltpu.sync_copy(dst_hbm.at[wid], didx)
        base = wid * (nch * CHUNK)

        def body(j, carry):
            cp1 = pltpu.async_copy(a_hbm.at[sidx.at[j]], r1, sem1)
            cp2 = pltpu.async_copy(st_hbm.at[sidx.at[j]], r2, sem2)
            cp3 = pltpu.async_copy(st_hbm.at[didx.at[j]], r3, sem3)
            cp1.wait()
            cp2.wait()
            cp3.wait()
            pltpu.sync_copy(r1, g1_hbm.at[pl.ds(base + j * CHUNK, CHUNK)])
            pltpu.sync_copy(r2, g2_hbm.at[pl.ds(base + j * CHUNK, CHUNK)])
            pltpu.sync_copy(r3, g3_hbm.at[pl.ds(base + j * CHUNK, CHUNK)])
            return carry

        lax.fori_loop(0, nch, body, 0)

    return gather_k


def _make_scatter(nch):
    mesh = plsc.VectorSubcoreMesh(core_axis_name="c", subcore_axis_name="s")
    rows_t = N_PAD // 16

    @functools.partial(
        pl.kernel,
        mesh=mesh,
        out_type=jax.ShapeDtypeStruct((2, N_PAD, MSG), f32),
        scratch_types=[
            pltpu.VMEM((nch, CHUNK), jnp.int32),
            pltpu.VMEM((CHUNK, MSG), f32),
            pltpu.VMEM_SHARED((N_PAD, MSG), f32),
        ],
        compiler_params=pltpu.CompilerParams(use_tc_tiling_on_sc=False),
    )
    def scatter_k(h2_hbm, dst_hbm, zero_hbm, out_hbm, didx, dat, acc):
        cid = lax.axis_index("c")
        sid = lax.axis_index("s")
        wid = sid * 2 + cid
        pltpu.sync_copy(zero_hbm.at[pl.ds(sid * rows_t, rows_t)],
                        acc.at[pl.ds(sid * rows_t, rows_t)])
        pltpu.sync_copy(dst_hbm.at[wid], didx)
        plsc.subcore_barrier()
        base = wid * (nch * CHUNK)

        def body(j, carry):
            pltpu.sync_copy(h2_hbm.at[pl.ds(base + j * CHUNK, CHUNK)], dat)
            pltpu.sync_copy(dat, acc.at[didx.at[j]], add=True)
            return carry

        lax.fori_loop(0, nch, body, 0)
        plsc.subcore_barrier()
        pltpu.sync_copy(acc.at[pl.ds(sid * rows_t, rows_t)],
                        out_hbm.at[cid, pl.ds(sid * rows_t, rows_t)])

    return scatter_k


def kernel(states, edge_index, x_init, phi_W1, phi_b1, phi_W2, phi_b2, phi_W3,
           phi_b3, lstm_Wih0, lstm_Whh0, lstm_bih0, lstm_bhh0,
           lstm_Wih1, lstm_Whh1, lstm_bih1, lstm_bhh1, out_W, out_b):
    e = edge_index.shape[1]
    nch = -(-e // (NW * CHUNK))
    e_pad = NW * nch * CHUNK

    pad_idx = jnp.full((e_pad - e,), N, jnp.int32)
    srcw = jnp.concatenate([edge_index[0], pad_idx]).reshape(NW, nch, CHUNK)
    dstw = jnp.concatenate([edge_index[1], pad_idx]).reshape(NW, nch, CHUNK)
    zeros_tab = jnp.zeros((N_PAD, MSG), f32)
    states_p = jnp.pad(states, ((0, N_PAD - N), (0, 0)))

    gather_k = _make_gather(e_pad, nch)
    scatter_k = _make_scatter(nch)

    x = x_init
    for l in range(2):
        relu_x = (l == 1)
        w1 = phi_W1[l]
        w1x = jnp.pad(w1[:FEAT], ((0, 0), (0, HID - 20)))
        w1s = jnp.pad(w1[FEAT:], ((0, 0), (0, HID - 20)))
        b1p = jnp.pad(phi_b1[l], (0, HID - 20)).reshape(1, HID)
        w2p = jnp.pad(phi_W2[l], ((0, HID - 20), (0, HID - 20)))
        b2p = jnp.pad(phi_b2[l], (0, HID - 20)).reshape(1, HID)
        w3p = jnp.pad(phi_W3[l], ((0, HID - 20), (0, 0)))
        b3p = phi_b3[l].reshape(1, FEAT)
        wih0t = lstm_Wih0[l].T
        wta = wih0t[:FEAT]
        wtx = wih0t[FEAT:]
        b0s = (lstm_bih0[l] + lstm_bhh0[l]).reshape(1, 4 * H)
        wh0 = lstm_Whh0[l].T
        wi1 = lstm_Wih1[l].T
        wh1 = lstm_Whh1[l].T
        b1s = (lstm_bih1[l] + lstm_bhh1[l]).reshape(1, 4 * H)

        x_p = jnp.pad(x, ((0, N_PAD - N), (0, 0)))
        a_t = _tables_call(x_p, w1x, relu_x)
        g1, g2, g3 = gather_k(a_t, states_p, srcw, dstw)
        h2 = _edge_mlp_call(g1, g2, g3, w1s, b1p, w2p, b2p, w3p)
        s2 = scatter_k(h2, dstw, zeros_tab)
        pre0 = _pre_call(s2[0, :N], s2[1, :N], x, b3p, wta, wtx, b0s, relu_x)
        x = _scan_call(pre0, wh0, wi1, wh1, b1s)

    wp = jnp.pad(out_W, ((0, 0), (0, FEAT - 1)))
    bp = jnp.broadcast_to(out_b.reshape(1, 1), (1, FEAT))
    y = _head_call(x, wp, bp)
    return y[:, :1]
